# baseline (speedup 1.0000x reference)
import functools

import jax
import jax.numpy as jnp
from jax import lax
from jax.experimental import pallas as pl
from jax.experimental.pallas import tpu as pltpu, tpu_sc as plsc

_VOCAB = 1000000
_D = 64
_LTOK = 200
_LPAD = 208
_B = 4096
_NC = 2
_NS = 16
_NW = _NC * _NS
_RPW = _B // _NW
_HALF = _LPAD // 2
_GROUPS = _LPAD // 16


def _rsqrt_newton(x):
  i = lax.bitcast_convert_type(x, jnp.int32)
  i = jnp.int32(0x5F3759DF) - lax.shift_right_logical(i, 1)
  y = lax.bitcast_convert_type(i, jnp.float32)
  for _ in range(3):
    y = y * (jnp.float32(1.5) - jnp.float32(0.5) * x * y * y)
  return y


def _sc_bow_body(x_hbm, emb_hbm, wt_hbm, out_hbm, idx_v, emb_v, wt_v, out_v,
                 sem):
  wid = lax.axis_index("s") * _NC + lax.axis_index("c")
  base = wid * _RPW
  pltpu.sync_copy(x_hbm.at[pl.ds(base * _LPAD, _RPW * _LPAD)], idx_v)

  def row_body(r, _):
    roff = r * _LPAD
    cp0 = pltpu.async_copy(emb_hbm.at[idx_v.at[pl.ds(roff, _HALF)]],
                           emb_v.at[pl.ds(0, _HALF), :], sem)
    cp1 = pltpu.async_copy(emb_hbm.at[idx_v.at[pl.ds(roff + _HALF, _HALF)]],
                           emb_v.at[pl.ds(_HALF, _HALF), :], sem)
    cp2 = pltpu.async_copy(wt_hbm.at[idx_v.at[pl.ds(roff, _HALF)]],
                           wt_v.at[pl.ds(0, _HALF)], sem)
    cp3 = pltpu.async_copy(wt_hbm.at[idx_v.at[pl.ds(roff + _HALF, _HALF)]],
                           wt_v.at[pl.ds(_HALF, _HALF)], sem)
    cp0.wait()
    cp1.wait()
    cp2.wait()
    cp3.wait()

    def group_body(g, carry):
      acc0, acc1, acc2, acc3, cnt = carry
      tok = idx_v[pl.ds(roff + g * 16, 16)]
      wv = wt_v[pl.ds(g * 16, 16)]
      maskv = tok != 0
      cnt = cnt + jnp.sum(jnp.where(maskv, jnp.float32(1.0), jnp.float32(0.0)))
      sig = jnp.float32(1.0) / (jnp.float32(1.0) + jnp.exp(-wv))
      scale_v = jnp.where(maskv, sig, jnp.float32(0.0))
      for t in range(16):
        e0 = emb_v[g * 16 + t, pl.ds(0, 16)]
        e1 = emb_v[g * 16 + t, pl.ds(16, 16)]
        e2 = emb_v[g * 16 + t, pl.ds(32, 16)]
        e3 = emb_v[g * 16 + t, pl.ds(48, 16)]
        ss = jnp.sum(e0 * e0 + e1 * e1 + e2 * e2 + e3 * e3)
        ss = jnp.maximum(ss, jnp.float32(1e-12))
        sc = scale_v[t] * _rsqrt_newton(ss)
        acc0 = acc0 + e0 * sc
        acc1 = acc1 + e1 * sc
        acc2 = acc2 + e2 * sc
        acc3 = acc3 + e3 * sc
      return acc0, acc1, acc2, acc3, cnt

    z = jnp.zeros((16,), jnp.float32)
    acc0, acc1, acc2, acc3, cnt = lax.fori_loop(
        0, _GROUPS, group_body, (z, z, z, z, jnp.float32(0.0)))
    inv = jnp.ones((16,), jnp.float32) / jnp.full((16,), cnt, jnp.float32)
    ooff = r * _D
    out_v[pl.ds(ooff, 16)] = acc0 * inv
    out_v[pl.ds(ooff + 16, 16)] = acc1 * inv
    out_v[pl.ds(ooff + 32, 16)] = acc2 * inv
    out_v[pl.ds(ooff + 48, 16)] = acc3 * inv
    return 0

  lax.fori_loop(0, _RPW, row_body, 0)
  pltpu.sync_copy(out_v, out_hbm.at[pl.ds(base * _D, _RPW * _D)])


@functools.partial(jax.jit)
def _sc_bow(xp, emb_table, wt_flat):
  mesh = plsc.VectorSubcoreMesh(core_axis_name="c", subcore_axis_name="s")
  return pl.kernel(
      _sc_bow_body,
      out_type=jax.ShapeDtypeStruct((_B * _D,), jnp.float32),
      mesh=mesh,
      scratch_types=[
          pltpu.VMEM((_RPW * _LPAD,), jnp.int32),
          pltpu.VMEM((_LPAD, _D), jnp.float32),
          pltpu.VMEM((_LPAD,), jnp.float32),
          pltpu.VMEM((_RPW * _D,), jnp.float32),
          pltpu.SemaphoreType.DMA,
      ],
      compiler_params=pltpu.CompilerParams(
          use_tc_tiling_on_sc=False, needs_layout_passes=False),
  )(xp, emb_table, wt_flat)


def _mlp_body(xb_ref, w1_ref, b1_ref, w2_ref, b2_ref, wc_ref, bc_ref, out_ref):
  x = xb_ref[...]
  h = jnp.maximum(
      jnp.dot(x, w1_ref[...], preferred_element_type=jnp.float32) + b1_ref[...],
      0.0)
  h = jnp.maximum(
      jnp.dot(h, w2_ref[...], preferred_element_type=jnp.float32) + b2_ref[...],
      0.0)
  logits = (
      jnp.dot(h, wc_ref[...], preferred_element_type=jnp.float32) + bc_ref[...])
  m = jnp.max(logits, axis=-1, keepdims=True)
  e = jnp.exp(logits - m)
  out_ref[...] = e / jnp.sum(e, axis=-1, keepdims=True)


@jax.jit
def _mlp(xbow, W1, b1, W2, b2, Wc, bc):
  bt = 512
  grid = (_B // bt,)
  return pl.pallas_call(
      _mlp_body,
      grid=grid,
      in_specs=[
          pl.BlockSpec((bt, _D), lambda i: (i, 0)),
          pl.BlockSpec((_D, 50), lambda i: (0, 0)),
          pl.BlockSpec((1, 50), lambda i: (0, 0)),
          pl.BlockSpec((50, 50), lambda i: (0, 0)),
          pl.BlockSpec((1, 50), lambda i: (0, 0)),
          pl.BlockSpec((50, 2), lambda i: (0, 0)),
          pl.BlockSpec((1, 2), lambda i: (0, 0)),
      ],
      out_specs=pl.BlockSpec((bt, 2), lambda i: (i, 0)),
      out_shape=jax.ShapeDtypeStruct((_B, 2), jnp.float32),
  )(xbow, W1, b1.reshape(1, 50), W2, b2.reshape(1, 50), Wc, bc.reshape(1, 2))


def kernel(X_input, emb_table, emb_weight_table, W1, b1, W2, b2, Wc, bc):
  xp = jnp.pad(X_input.astype(jnp.int32),
               ((0, 0), (0, _LPAD - _LTOK))).reshape((_B * _LPAD,))
  wt_flat = emb_weight_table.reshape((_VOCAB,))
  xbow = _sc_bow(xp, emb_table, wt_flat).reshape((_B, _D))
  return _mlp(xbow, W1, b1, W2, b2, Wc, bc)
